# baseline (speedup 1.0000x reference)
import functools

import jax
import jax.numpy as jnp
from jax import lax
from jax.experimental import pallas as pl
from jax.experimental.pallas import tpu as pltpu
from jax.experimental.pallas import tpu_sc as plsc

K = 6000
N = 20000
NPAD = 20480
R, C = 256, 128
NSORT = R * C
KPAD = 6144
D = 256
DA = 128

_NC, _NS = 2, 16
_NW = _NC * _NS
_BPW = KPAD // _NW


def _stage(v, ix, j, k, flip=False):
    rows = v.shape[0]
    row = lax.broadcasted_iota(jnp.int32, (rows, C), 0)
    lane = lax.broadcasted_iota(jnp.int32, (rows, C), 1)
    i = row * C + lane
    if j < C:
        pm = jnp.roll(v, -j, axis=1)
        pp = jnp.roll(v, j, axis=1)
        qm = jnp.roll(ix, -j, axis=1)
        qp = jnp.roll(ix, j, axis=1)
    else:
        J = j // C
        pm = jnp.roll(v, -J, axis=0)
        pp = jnp.roll(v, J, axis=0)
        qm = jnp.roll(ix, -J, axis=0)
        qp = jnp.roll(ix, J, axis=0)
    lower = (i & j) == 0
    pv = jnp.where(lower, pm, pp)
    pix = jnp.where(lower, qm, qp)
    dir_desc = (i & k) == 0
    if flip:
        dir_desc = ~dir_desc
    w = (v > pv) | ((v == pv) & (ix < pix))
    keep_mine = (lower == dir_desc) == w
    return jnp.where(keep_mine, v, pv), jnp.where(keep_mine, ix, pix)


def _sort_upto(v, ix, kmax, flip_at=None):
    k = 2
    while k <= kmax:
        j = k // 2
        while j >= 1:
            v, ix = _stage(v, ix, j, k, flip=(k == flip_at))
            j //= 2
        k *= 2
    return v, ix


def _merge(v, ix, k, jmax, flip=False):
    j = jmax
    while j >= 1:
        v, ix = _stage(v, ix, j, k, flip=flip)
        j //= 2
    return v, ix


def _topk_sort_body(conf_ref, anc_ref, val_ref, idx_ref, anc_out_ref):
    anc_out_ref[:, :] = jnp.pad(anc_ref[:, :], ((0, 0), (0, DA - 11)))

    x = conf_ref[:, :]
    m = jnp.max(x, axis=0)
    m2 = m.reshape(NPAD // C, C)

    vA = m2[:128]
    ixA = lax.broadcasted_iota(jnp.int32, (128, C), 0) * C + \
        lax.broadcasted_iota(jnp.int32, (128, C), 1)
    vB = m2[128:160]
    ixB = ixA[:32] + (128 * C)

    vA, ixA = _sort_upto(vA, ixA, 4096)
    vB, ixB = _sort_upto(vB, ixB, 4096)

    vA, ixA = _merge(vA, ixA, 8192, 4096)
    dead1 = jnp.full((32, C), -jnp.inf, jnp.float32)
    ixd1 = lax.broadcasted_iota(jnp.int32, (32, C), 0) * C + \
        lax.broadcasted_iota(jnp.int32, (32, C), 1) + (160 * C)
    vB = jnp.concatenate([vB, dead1], axis=0)
    ixB = jnp.concatenate([ixB, ixd1], axis=0)
    vB, ixB = _merge(vB, ixB, 8192, 4096)

    vA, ixA = _merge(vA, ixA, 16384, 8192)
    dead2 = jnp.full((64, C), -jnp.inf, jnp.float32)
    ixd2 = lax.broadcasted_iota(jnp.int32, (64, C), 0) * C + \
        lax.broadcasted_iota(jnp.int32, (64, C), 1) + (192 * C)
    vB = jnp.concatenate([vB, dead2], axis=0)
    ixB = jnp.concatenate([ixB, ixd2], axis=0)
    vB, ixB = _merge(vB, ixB, 16384, 8192, flip=True)

    w = (vA > vB) | ((vA == vB) & (ixA < ixB))
    vA = jnp.where(w, vA, vB)
    ixA = jnp.where(w, ixA, ixB)
    vA, ixA = _stage(vA, ixA, 8192, NSORT)
    vA = vA[:64]
    ixA = ixA[:64]
    vA, ixA = _merge(vA, ixA, NSORT, 4096)

    vtop = vA[: KPAD // C]
    val_ref[:, :] = 1.0 / (1.0 + jnp.exp(-vtop))
    idx_ref[:, :] = ixA[: KPAD // C]


_topk_sort = pl.pallas_call(
    _topk_sort_body,
    out_shape=(
        jax.ShapeDtypeStruct((KPAD // C, C), jnp.float32),
        jax.ShapeDtypeStruct((KPAD // C, C), jnp.int32),
        jax.ShapeDtypeStruct((N, DA), jnp.float32),
    ),
)


def _gather_body(feat_hbm, anc_hbm, idx_hbm, out_f, out_a,
                 idx_a, idx_b, rows_fa, rows_fb, rows_aa, rows_ab, sem, wsem):
    wid = lax.axis_index("s") * _NC + lax.axis_index("c")
    base = wid * _BPW

    @pl.when(wid < _NW - 1)
    def _full_chunk():
        pltpu.sync_copy(idx_hbm.at[pl.ds(base, 128)], idx_a)
        pltpu.sync_copy(idx_hbm.at[pl.ds(base + 128, 64)], idx_b)
        c1 = pltpu.async_copy(feat_hbm.at[idx_a], rows_fa, sem)
        c2 = pltpu.async_copy(feat_hbm.at[idx_b], rows_fb, sem)
        c3 = pltpu.async_copy(anc_hbm.at[idx_a], rows_aa, sem)
        c4 = pltpu.async_copy(anc_hbm.at[idx_b], rows_ab, sem)
        c1.wait()
        w1 = pltpu.async_copy(rows_fa, out_f.at[pl.ds(base, 128)], wsem)
        c2.wait()
        w2 = pltpu.async_copy(rows_fb, out_f.at[pl.ds(base + 128, 64)], wsem)
        c3.wait()
        w3 = pltpu.async_copy(rows_aa, out_a.at[pl.ds(base, 128)], wsem)
        c4.wait()
        w4 = pltpu.async_copy(rows_ab, out_a.at[pl.ds(base + 128, 64)], wsem)
        w1.wait()
        w2.wait()
        w3.wait()
        w4.wait()

    @pl.when(wid == _NW - 1)
    def _tail_chunk():
        pltpu.sync_copy(idx_hbm.at[pl.ds(base, 48)], idx_a.at[pl.ds(0, 48)])
        c1 = pltpu.async_copy(feat_hbm.at[idx_a.at[pl.ds(0, 48)]],
                              rows_fa.at[pl.ds(0, 48)], sem)
        c2 = pltpu.async_copy(anc_hbm.at[idx_a.at[pl.ds(0, 48)]],
                              rows_aa.at[pl.ds(0, 48)], sem)
        c1.wait()
        c2.wait()
        pltpu.sync_copy(rows_fa.at[pl.ds(0, 48)], out_f.at[pl.ds(base, 48)])
        pltpu.sync_copy(rows_aa.at[pl.ds(0, 48)], out_a.at[pl.ds(base, 48)])


@functools.cache
def _make_gather():
  return pl.kernel(
    _gather_body,
    out_type=(
        jax.ShapeDtypeStruct((K, D), jnp.float32),
        jax.ShapeDtypeStruct((K, DA), jnp.float32),
    ),
    mesh=plsc.VectorSubcoreMesh(core_axis_name="c", subcore_axis_name="s",
                                num_cores=_NC, num_subcores=_NS),
    scratch_types=[
        pltpu.VMEM((128,), jnp.int32),
        pltpu.VMEM((64,), jnp.int32),
        pltpu.VMEM((128, D), jnp.float32),
        pltpu.VMEM((64, D), jnp.float32),
        pltpu.VMEM((128, DA), jnp.float32),
        pltpu.VMEM((64, DA), jnp.float32),
        pltpu.SemaphoreType.DMA,
        pltpu.SemaphoreType.DMA,
    ],
  )


def kernel(instance_feature, anchor, confidence):
    conf_t = jnp.pad(confidence[0].T, ((0, 6), (0, NPAD - N)),
                     constant_values=-jnp.inf)
    vals2d, idx2d, anc_pad = _topk_sort(conf_t, anchor[0])
    idx_flat = idx2d.reshape(KPAD)
    feat_flat = instance_feature.reshape(8 * N, D)
    feat_sel, anc_sel = _make_gather()(feat_flat, anc_pad, idx_flat)
    top_conf = vals2d.reshape(KPAD)[:K][None]
    return (top_conf, feat_sel[None], anc_sel[:, :11][None])


# speedup vs baseline: 1.0290x; 1.0290x over previous
import functools

import jax
import jax.numpy as jnp
from jax import lax
from jax.experimental import pallas as pl
from jax.experimental.pallas import tpu as pltpu
from jax.experimental.pallas import tpu_sc as plsc

K = 6000
N = 20000
NPAD = 20480
R, C = 256, 128
NSORT = R * C
KPAD = 6144
D = 256
DA = 128

_NC, _NS = 2, 16
_NW = _NC * _NS
_BPW = KPAD // _NW


def _stage(v, ix, j, k, flip=False):
    rows = v.shape[0]
    row = lax.broadcasted_iota(jnp.int32, (rows, C), 0)
    lane = lax.broadcasted_iota(jnp.int32, (rows, C), 1)
    i = row * C + lane
    if j < C:
        pm = jnp.roll(v, -j, axis=1)
        pp = jnp.roll(v, j, axis=1)
        qm = jnp.roll(ix, -j, axis=1)
        qp = jnp.roll(ix, j, axis=1)
    else:
        J = j // C
        pm = jnp.roll(v, -J, axis=0)
        pp = jnp.roll(v, J, axis=0)
        qm = jnp.roll(ix, -J, axis=0)
        qp = jnp.roll(ix, J, axis=0)
    lower = (i & j) == 0
    pv = jnp.where(lower, pm, pp)
    pix = jnp.where(lower, qm, qp)
    dir_desc = (i & k) == 0
    if flip:
        dir_desc = ~dir_desc
    w = (v > pv) | ((v == pv) & (ix < pix))
    keep_mine = (lower == dir_desc) == w
    return jnp.where(keep_mine, v, pv), jnp.where(keep_mine, ix, pix)


def _sort_upto(v, ix, kmax, flip_at=None):
    k = 2
    while k <= kmax:
        j = k // 2
        while j >= 1:
            v, ix = _stage(v, ix, j, k, flip=(k == flip_at))
            j //= 2
        k *= 2
    return v, ix


def _merge(v, ix, k, jmax, flip=False):
    j = jmax
    while j >= 1:
        v, ix = _stage(v, ix, j, k, flip=flip)
        j //= 2
    return v, ix


def _topk_sort_body(conf_ref, val_ref, idx_ref):
    x = conf_ref[:, :]
    m = jnp.max(x, axis=0)
    m2 = m.reshape(NPAD // C, C)

    vA = m2[:128]
    ixA = lax.broadcasted_iota(jnp.int32, (128, C), 0) * C + \
        lax.broadcasted_iota(jnp.int32, (128, C), 1)
    vB = m2[128:160]
    ixB = ixA[:32] + (128 * C)

    vA, ixA = _sort_upto(vA, ixA, 4096)
    vB, ixB = _sort_upto(vB, ixB, 4096)

    vA, ixA = _merge(vA, ixA, 8192, 4096)
    dead1 = jnp.full((32, C), -jnp.inf, jnp.float32)
    ixd1 = lax.broadcasted_iota(jnp.int32, (32, C), 0) * C + \
        lax.broadcasted_iota(jnp.int32, (32, C), 1) + (160 * C)
    vB = jnp.concatenate([vB, dead1], axis=0)
    ixB = jnp.concatenate([ixB, ixd1], axis=0)
    vB, ixB = _merge(vB, ixB, 8192, 4096)

    vA, ixA = _merge(vA, ixA, 16384, 8192)
    dead2 = jnp.full((64, C), -jnp.inf, jnp.float32)
    ixd2 = lax.broadcasted_iota(jnp.int32, (64, C), 0) * C + \
        lax.broadcasted_iota(jnp.int32, (64, C), 1) + (192 * C)
    vB = jnp.concatenate([vB, dead2], axis=0)
    ixB = jnp.concatenate([ixB, ixd2], axis=0)
    vB, ixB = _merge(vB, ixB, 16384, 8192, flip=True)

    w = (vA > vB) | ((vA == vB) & (ixA < ixB))
    vA = jnp.where(w, vA, vB)
    ixA = jnp.where(w, ixA, ixB)
    vA, ixA = _stage(vA, ixA, 8192, NSORT)
    vA = vA[:64]
    ixA = ixA[:64]
    vA, ixA = _merge(vA, ixA, NSORT, 4096)

    vtop = vA[: KPAD // C]
    val_ref[:, :] = 1.0 / (1.0 + jnp.exp(-vtop))
    idx_ref[:, :] = ixA[: KPAD // C]


_topk_sort = pl.pallas_call(
    _topk_sort_body,
    out_shape=(
        jax.ShapeDtypeStruct((KPAD // C, C), jnp.float32),
        jax.ShapeDtypeStruct((KPAD // C, C), jnp.int32),
    ),
)


def _gather_body(feat_hbm, anc_hbm, idx_hbm, out_f, out_a,
                 idx_a, idx_b, rows_fa, rows_fb, rows_aa, rows_ab, sem, wsem):
    wid = lax.axis_index("s") * _NC + lax.axis_index("c")
    base = wid * _BPW

    @pl.when(wid < _NW - 1)
    def _full_chunk():
        pltpu.sync_copy(idx_hbm.at[pl.ds(base, 128)], idx_a)
        pltpu.sync_copy(idx_hbm.at[pl.ds(base + 128, 64)], idx_b)
        c1 = pltpu.async_copy(feat_hbm.at[idx_a], rows_fa, sem)
        c2 = pltpu.async_copy(feat_hbm.at[idx_b], rows_fb, sem)
        c3 = pltpu.async_copy(anc_hbm.at[idx_a], rows_aa, sem)
        c4 = pltpu.async_copy(anc_hbm.at[idx_b], rows_ab, sem)
        c1.wait()
        w1 = pltpu.async_copy(rows_fa, out_f.at[pl.ds(base, 128)], wsem)
        c2.wait()
        w2 = pltpu.async_copy(rows_fb, out_f.at[pl.ds(base + 128, 64)], wsem)
        c3.wait()
        w3 = pltpu.async_copy(rows_aa, out_a.at[pl.ds(base, 128)], wsem)
        c4.wait()
        w4 = pltpu.async_copy(rows_ab, out_a.at[pl.ds(base + 128, 64)], wsem)
        w1.wait()
        w2.wait()
        w3.wait()
        w4.wait()

    @pl.when(wid == _NW - 1)
    def _tail_chunk():
        pltpu.sync_copy(idx_hbm.at[pl.ds(base, 48)], idx_a.at[pl.ds(0, 48)])
        c1 = pltpu.async_copy(feat_hbm.at[idx_a.at[pl.ds(0, 48)]],
                              rows_fa.at[pl.ds(0, 48)], sem)
        c2 = pltpu.async_copy(anc_hbm.at[idx_a.at[pl.ds(0, 48)]],
                              rows_aa.at[pl.ds(0, 48)], sem)
        c1.wait()
        c2.wait()
        pltpu.sync_copy(rows_fa.at[pl.ds(0, 48)], out_f.at[pl.ds(base, 48)])
        pltpu.sync_copy(rows_aa.at[pl.ds(0, 48)], out_a.at[pl.ds(base, 48)])


@functools.cache
def _make_gather():
  return pl.kernel(
    _gather_body,
    out_type=(
        jax.ShapeDtypeStruct((K, D), jnp.float32),
        jax.ShapeDtypeStruct((K, DA), jnp.float32),
    ),
    mesh=plsc.VectorSubcoreMesh(core_axis_name="c", subcore_axis_name="s",
                                num_cores=_NC, num_subcores=_NS),
    scratch_types=[
        pltpu.VMEM((128,), jnp.int32),
        pltpu.VMEM((64,), jnp.int32),
        pltpu.VMEM((128, D), jnp.float32),
        pltpu.VMEM((64, D), jnp.float32),
        pltpu.VMEM((128, DA), jnp.float32),
        pltpu.VMEM((64, DA), jnp.float32),
        pltpu.SemaphoreType.DMA,
        pltpu.SemaphoreType.DMA,
    ],
  )


def kernel(instance_feature, anchor, confidence):
    conf_t = jnp.pad(confidence[0].T, ((0, 6), (0, NPAD - N)),
                     constant_values=-jnp.inf)
    vals2d, idx2d = _topk_sort(conf_t)
    idx_flat = idx2d.reshape(KPAD)
    anc_pad = jnp.pad(anchor[0], ((0, 0), (0, DA - 11)))
    feat_flat = instance_feature.reshape(8 * N, D)
    feat_sel, anc_sel = _make_gather()(feat_flat, anc_pad, idx_flat)
    top_conf = vals2d.reshape(KPAD)[:K][None]
    return (top_conf, feat_sel[None], anc_sel[:, :11][None])
